# baseline (speedup 1.0000x reference)
import functools

import jax
import jax.numpy as jnp
from jax import lax
from jax.experimental import pallas as pl
from jax.experimental.pallas import tpu as pltpu
from jax.experimental.pallas import tpu_sc as plsc

_VOCAB = 1000000
_EMBED_DIM = 16
_BATCH = 16384

_NUM_CORES = 2
_NUM_SUBCORES = 16
_NUM_WORKERS = _NUM_CORES * _NUM_SUBCORES
_B_PER_W = _BATCH // _NUM_WORKERS
_LANES = 16
_CHUNK = 256
_NCHUNKS = _B_PER_W // _CHUNK
_CGROUPS = _CHUNK // _LANES
_ROWS128 = _VOCAB * _EMBED_DIM // 128


@functools.partial(
    pl.kernel,
    mesh=plsc.VectorSubcoreMesh(core_axis_name="c", subcore_axis_name="s"),
    out_type=jax.ShapeDtypeStruct((_BATCH, _EMBED_DIM), jnp.float32),
    scratch_types=[
        pltpu.VMEM((_B_PER_W,), jnp.int32),
        pltpu.VMEM((_CHUNK,), jnp.int32),
        pltpu.VMEM((_B_PER_W,), jnp.int32),
        pltpu.VMEM((_CHUNK, 128), jnp.float32),
        pltpu.VMEM((_B_PER_W, _EMBED_DIM), jnp.float32),
        pltpu.SemaphoreType.DMA,
    ],
    compiler_params=pltpu.CompilerParams(needs_layout_passes=False),
)
def _gather_kernel(table_hbm, idx_hbm, out_hbm, idx_v, qidx_v, off_v,
                   rows_v, out_v, sem):
    wid = lax.axis_index("s") * _NUM_CORES + lax.axis_index("c")
    base = wid * _B_PER_W
    pltpu.sync_copy(idx_hbm.at[pl.ds(base, _B_PER_W)], idx_v)

    for c in range(_NCHUNKS):
        cbase = c * _CHUNK

        def _prep(g, carry):
            v = idx_v[pl.ds(cbase + g * _LANES, _LANES)]
            qidx_v[pl.ds(g * _LANES, _LANES)] = lax.shift_right_logical(v, 3)
            off_v[pl.ds(cbase + g * _LANES, _LANES)] = lax.shift_left(v & 7, 4)
            return carry

        lax.fori_loop(0, _CGROUPS, _prep, 0)

        pltpu.async_copy(table_hbm.at[qidx_v], rows_v, sem).wait()

        def _extract(g, carry):
            row_ids = lax.iota(jnp.int32, _LANES) + g * _LANES
            off = off_v[pl.ds(cbase + g * _LANES, _LANES)]
            for j in range(_EMBED_DIM):
                vals = plsc.load_gather(rows_v, [row_ids, off + j])
                col = jnp.full((_LANES,), j, jnp.int32)
                plsc.store_scatter(out_v, [cbase + row_ids, col], vals)
            return carry

        lax.fori_loop(0, _CGROUPS, _extract, 0)

    pltpu.sync_copy(out_v, out_hbm.at[pl.ds(base, _B_PER_W)])


def kernel(inputs, table):
    table128 = table.reshape(_ROWS128, 128)
    return _gather_kernel(table128, inputs.astype(jnp.int32))


# speedup vs baseline: 1.6944x; 1.6944x over previous
import functools

import jax
import jax.numpy as jnp
from jax import lax
from jax.experimental import pallas as pl
from jax.experimental.pallas import tpu as pltpu
from jax.experimental.pallas import tpu_sc as plsc

_VOCAB = 1000000
_EMBED_DIM = 16
_BATCH = 16384

_NUM_CORES = 2
_NUM_SUBCORES = 16
_NUM_WORKERS = _NUM_CORES * _NUM_SUBCORES
_B_PER_W = _BATCH // _NUM_WORKERS
_LANES = 16
_GROUPS = _B_PER_W // _LANES


@functools.partial(
    pl.kernel,
    mesh=plsc.VectorSubcoreMesh(core_axis_name="c", subcore_axis_name="s"),
    out_type=jax.ShapeDtypeStruct((_BATCH, _EMBED_DIM), jnp.float32),
    scratch_types=[
        pltpu.VMEM((_B_PER_W,), jnp.int32),
        pltpu.VMEM((_B_PER_W, _EMBED_DIM), jnp.float32),
        pltpu.SemaphoreType.DMA,
    ],
    compiler_params=pltpu.CompilerParams(needs_layout_passes=False),
)
def _gather_kernel(table_hbm, idx_hbm, out_hbm, idx_v, out_v, sem):
    wid = lax.axis_index("s") * _NUM_CORES + lax.axis_index("c")
    base = wid * _B_PER_W
    pltpu.sync_copy(idx_hbm.at[pl.ds(base, _B_PER_W)], idx_v)

    lanes = lax.iota(jnp.int32, _LANES)

    def _body(g, carry):
        v = idx_v[pl.ds(g * _LANES, _LANES)]
        for j in range(_LANES):
            s = jnp.max(jnp.where(lanes == j, v, 0))
            pltpu.async_copy(
                table_hbm.at[pl.ds(s, 1)],
                out_v.at[pl.ds(g * _LANES + j, 1)],
                sem,
            )
        return carry

    lax.fori_loop(0, _GROUPS, _body, 0)

    pltpu.make_async_copy(
        table_hbm.at[pl.ds(0, _B_PER_W)], out_v, sem
    ).wait()

    pltpu.sync_copy(out_v, out_hbm.at[pl.ds(base, _B_PER_W)])


def kernel(inputs, table):
    return _gather_kernel(table, inputs.astype(jnp.int32))
